# Initial kernel scaffold; baseline (speedup 1.0000x reference)
import jax
import jax.numpy as jnp
from jax.experimental import pallas as pl


def kernel(input, sizes):
    raise NotImplementedError("write your pallas kernel here")


# speedup vs baseline: 8.2442x; 8.2442x over previous
import jax
import jax.numpy as jnp
from jax.experimental import pallas as pl
from jax.experimental.pallas import tpu as pltpu

_BR = 512
_NROWS = 32768
_D = 1024
_B = 16


def _tc_body(ends_ref, x_ref, o_ref):
    i = pl.program_id(0)

    @pl.when(i == 0)
    def _init():
        o_ref[...] = jnp.full((_B, _D), -jnp.inf, jnp.float32)

    row0 = i * _BR
    row1 = row0 + _BR - 1
    s0 = jnp.int32(0)
    s1 = jnp.int32(0)
    for k in range(_B):
        e = ends_ref[k]
        s0 += (row0 >= e).astype(jnp.int32)
        s1 += (row1 >= e).astype(jnp.int32)

    @pl.when((s0 == s1) & (s0 < _B))
    def _fast():
        m = jnp.max(x_ref[...], axis=0, keepdims=True)
        cur = o_ref[pl.ds(s0, 1), :]
        o_ref[pl.ds(s0, 1), :] = jnp.maximum(cur, m)

    @pl.when((s0 != s1) & (s0 < _B))
    def _slow():
        rows = row0 + jax.lax.broadcasted_iota(jnp.int32, (_BR, 1), 0)
        x = x_ref[...]

        def body(s, carry):
            idx = jnp.maximum(s - 1, 0)
            start = jnp.where(s == 0, 0, ends_ref[idx])
            end = ends_ref[s]
            mask = (rows >= start) & (rows < end)
            m = jnp.max(jnp.where(mask, x, -jnp.inf), axis=0, keepdims=True)
            cur = o_ref[pl.ds(s, 1), :]
            o_ref[pl.ds(s, 1), :] = jnp.maximum(cur, m)
            return carry

        jax.lax.fori_loop(s0, jnp.minimum(s1, _B - 1) + 1, body, 0)


def kernel(input, sizes):
    ends = jnp.cumsum(sizes.astype(jnp.int32))
    grid_spec = pltpu.PrefetchScalarGridSpec(
        num_scalar_prefetch=1,
        grid=(_NROWS // _BR,),
        in_specs=[
            pl.BlockSpec(
                (_BR, _D),
                lambda i, ends: (
                    jnp.minimum(i, (ends[_B - 1] + _BR - 1) // _BR - 1), 0),
            )
        ],
        out_specs=pl.BlockSpec((_B, _D), lambda i, ends: (0, 0)),
    )
    return pl.pallas_call(
        _tc_body,
        grid_spec=grid_spec,
        out_shape=jax.ShapeDtypeStruct((_B, _D), jnp.float32),
        compiler_params=pltpu.CompilerParams(
            dimension_semantics=("arbitrary",)),
    )(ends, input)
